# Initial kernel scaffold; baseline (speedup 1.0000x reference)
import jax
import jax.numpy as jnp
from jax.experimental import pallas as pl


def kernel(net_weights, flat_nodepin, nodepin_start, pin2net_map, num_nodes):
    raise NotImplementedError("write your pallas kernel here")


# speedup vs baseline: 2062.4197x; 2062.4197x over previous
import functools

import jax
import jax.numpy as jnp
from jax import lax
from jax.experimental import pallas as pl
from jax.experimental.pallas import tpu as pltpu
from jax.experimental.pallas import tpu_sc as plsc

NC = 2
NS = 16
NW = NC * NS
L = 16

P = 6_400_000
SLOTS_PER_W = P // NW
CHUNK = 4000
N_CHUNKS = SLOTS_PER_W // CHUNK

NNETS = 100_000

NODES_PER_W = 3200
NPAD = NW * NODES_PER_W
NODE_CHUNK = 1600
SPAD_LEN = (NW - 1) * NODES_PER_W + NODE_CHUNK + 1608

_mesh = plsc.VectorSubcoreMesh(core_axis_name="c", subcore_axis_name="s")
_params = pltpu.CompilerParams(needs_layout_passes=False)


def _wid():
    return lax.axis_index("s") * NC + lax.axis_index("c")


@functools.partial(
    pl.kernel,
    out_type=(
        jax.ShapeDtypeStruct((P,), jnp.float32),
        jax.ShapeDtypeStruct((NW, L), jnp.float32),
    ),
    mesh=_mesh,
    compiler_params=_params,
    scratch_types=(
        pltpu.VMEM((NNETS,), jnp.float32),
        pltpu.VMEM((CHUNK,), jnp.int32),
        pltpu.VMEM((CHUNK,), jnp.int32),
        pltpu.VMEM((CHUNK,), jnp.float32),
        pltpu.VMEM((L,), jnp.float32),
        pltpu.SemaphoreType.DMA,
    ),
)
def _phase1(wt_hbm, fnp_hbm, p2n_hbm, cums_hbm, totals_hbm,
            wt_v, fnp_v, net_v, cums_v, tot_v, sem):
    wid = _wid()
    base = wid * SLOTS_PER_W
    pltpu.sync_copy(wt_hbm, wt_v)

    def chunk_body(c, carry):
        off = pl.multiple_of(base + c * CHUNK, 8)
        pltpu.sync_copy(fnp_hbm.at[pl.ds(off, CHUNK)], fnp_v)
        pltpu.async_copy(p2n_hbm.at[fnp_v], net_v, sem).wait()

        def vec_body(j, cin):
            idx = net_v[pl.ds(j * L, L)]
            w = plsc.load_gather(wt_v, [idx])
            cums_v[pl.ds(j * L, L)] = plsc.cumsum(w) + cin
            return cin + jnp.sum(w)

        carry = lax.fori_loop(0, CHUNK // L, vec_body, carry)
        pltpu.sync_copy(cums_v, cums_hbm.at[pl.ds(off, CHUNK)])
        return carry

    total = lax.fori_loop(0, N_CHUNKS, chunk_body, jnp.float32(0.0))
    tot_v[...] = jnp.full((L,), total, dtype=jnp.float32)
    pltpu.sync_copy(tot_v, totals_hbm.at[wid])


@functools.partial(
    pl.kernel,
    out_type=jax.ShapeDtypeStruct((NPAD,), jnp.float32),
    mesh=_mesh,
    compiler_params=_params,
    scratch_types=(
        pltpu.VMEM((NW,), jnp.float32),
        pltpu.VMEM((NW,), jnp.float32),
        pltpu.VMEM((1608,), jnp.int32),
        pltpu.VMEM((NODE_CHUNK,), jnp.int32),
        pltpu.VMEM((NODE_CHUNK,), jnp.int32),
        pltpu.VMEM((NODE_CHUNK,), jnp.float32),
        pltpu.VMEM((NODE_CHUNK,), jnp.float32),
        pltpu.VMEM((NODE_CHUNK,), jnp.float32),
        pltpu.SemaphoreType.DMA,
    ),
)
def _phase2(cums_hbm, tot32_hbm, start_hbm, out_hbm,
            tot_v, bases_v, sv, eidx, bidx, le_v, lb_v, ov, sem):
    wid = _wid()
    pltpu.sync_copy(tot32_hbm, tot_v)
    t0 = tot_v[pl.ds(0, L)]
    t1 = tot_v[pl.ds(L, L)]
    bases_v[pl.ds(0, L)] = plsc.cumsum(t0) - t0
    bases_v[pl.ds(L, L)] = plsc.cumsum(t1) - t1 + jnp.sum(t0)

    def chunk_body(c, _):
        n0 = pl.multiple_of(wid * NODES_PER_W + c * NODE_CHUNK, 8)
        pltpu.sync_copy(start_hbm.at[pl.ds(n0, 1608)], sv)

        def build(j, _):
            lo = sv[pl.ds(j * L, L)]
            hi = sv[pl.ds(j * L + 1, L)]
            eidx[pl.ds(j * L, L)] = jnp.maximum(hi - 1, 0)
            bidx[pl.ds(j * L, L)] = jnp.maximum(lo - 1, 0)
            return 0

        lax.fori_loop(0, NODE_CHUNK // L, build, 0)
        pltpu.async_copy(cums_hbm.at[eidx], le_v, sem).wait()
        pltpu.async_copy(cums_hbm.at[bidx], lb_v, sem).wait()

        def comp(j, _):
            lo = sv[pl.ds(j * L, L)]
            hi = sv[pl.ds(j * L + 1, L)]
            e = hi - 1
            b = lo - 1
            ec = jnp.maximum(e, 0)
            bc = jnp.maximum(b, 0)
            be = plsc.load_gather(bases_v, [ec // SLOTS_PER_W])
            bb = plsc.load_gather(bases_v, [bc // SLOTS_PER_W])
            ge = jnp.where(e < 0, jnp.float32(0.0), le_v[pl.ds(j * L, L)] + be)
            gb = jnp.where(b < 0, jnp.float32(0.0), lb_v[pl.ds(j * L, L)] + bb)
            ov[pl.ds(j * L, L)] = ge - gb
            return 0

        lax.fori_loop(0, NODE_CHUNK // L, comp, 0)
        pltpu.sync_copy(ov, out_hbm.at[pl.ds(n0, NODE_CHUNK)])
        return 0

    lax.fori_loop(0, NODES_PER_W // NODE_CHUNK, chunk_body, 0)


def kernel(net_weights, flat_nodepin, nodepin_start, pin2net_map, num_nodes):
    n = nodepin_start.shape[0] - 1
    p = flat_nodepin.shape[0]
    fnp = flat_nodepin.astype(jnp.int32)
    p2n = pin2net_map.astype(jnp.int32)
    start_pad = jnp.concatenate([
        nodepin_start.astype(jnp.int32),
        jnp.full((SPAD_LEN - (n + 1),), p, dtype=jnp.int32),
    ])
    cums, totals = _phase1(net_weights, fnp, p2n)
    out_pad = _phase2(cums, totals[:, 0], start_pad)
    return out_pad[:n]


# speedup vs baseline: 3209.3457x; 1.5561x over previous
import functools

import jax
import jax.numpy as jnp
from jax import lax
from jax.experimental import pallas as pl
from jax.experimental.pallas import tpu as pltpu
from jax.experimental.pallas import tpu_sc as plsc

NC = 2
NS = 16
NW = NC * NS
L = 16

P = 6_400_000
SLOTS_PER_W = P // NW
CHUNK = 4000
N_CHUNKS = SLOTS_PER_W // CHUNK

NNETS = 100_000

NODES_PER_W = 3200
NPAD = NW * NODES_PER_W
NODE_CHUNK = 1600
SPAD_LEN = (NW - 1) * NODES_PER_W + NODE_CHUNK + 1608

_mesh = plsc.VectorSubcoreMesh(core_axis_name="c", subcore_axis_name="s")
_params = pltpu.CompilerParams(needs_layout_passes=False)


def _wid():
    return lax.axis_index("s") * NC + lax.axis_index("c")


@functools.partial(
    pl.kernel,
    out_type=(
        jax.ShapeDtypeStruct((P,), jnp.float32),
        jax.ShapeDtypeStruct((NW, L), jnp.float32),
    ),
    mesh=_mesh,
    compiler_params=_params,
    scratch_types=(
        pltpu.VMEM((NNETS,), jnp.float32),
        pltpu.VMEM((CHUNK,), jnp.int32),
        pltpu.VMEM((CHUNK,), jnp.int32),
        pltpu.VMEM((CHUNK,), jnp.int32),
        pltpu.VMEM((CHUNK,), jnp.int32),
        pltpu.VMEM((CHUNK,), jnp.float32),
        pltpu.VMEM((L,), jnp.float32),
        pltpu.SemaphoreType.DMA,
        pltpu.SemaphoreType.DMA,
    ),
)
def _phase1(wt_hbm, fnp_hbm, p2n_hbm, cums_hbm, totals_hbm,
            wt_v, fnp0, fnp1, net0, net1, cums_v, tot_v, sem0, sem1):
    wid = _wid()
    base = wid * SLOTS_PER_W
    pltpu.sync_copy(wt_hbm, wt_v)

    fnp_b = (fnp0, fnp1)
    net_b = (net0, net1)
    sem_b = (sem0, sem1)

    def _prefetch(k, b):
        off = pl.multiple_of(base + k * CHUNK, 8)
        pltpu.sync_copy(fnp_hbm.at[pl.ds(off, CHUNK)], fnp_b[b])
        pltpu.async_copy(p2n_hbm.at[fnp_b[b]], net_b[b], sem_b[b])

    def _compute(k, b, carry):
        pltpu.make_async_copy(p2n_hbm.at[fnp_b[b]], net_b[b], sem_b[b]).wait()

        def vec_body(j, cin):
            idx = net_b[b][pl.ds(j * L, L)]
            w = plsc.load_gather(wt_v, [idx])
            cums_v[pl.ds(j * L, L)] = plsc.cumsum(w) + cin
            return cin + jnp.sum(w)

        carry = lax.fori_loop(0, CHUNK // L, vec_body, carry, unroll=4)
        off = pl.multiple_of(base + k * CHUNK, 8)
        pltpu.sync_copy(cums_v, cums_hbm.at[pl.ds(off, CHUNK)])
        return carry

    _prefetch(0, 0)

    def body(c2, carry):
        k0 = 2 * c2
        _prefetch(k0 + 1, 1)
        carry = _compute(k0, 0, carry)

        @pl.when(c2 < N_CHUNKS // 2 - 1)
        def _():
            _prefetch(k0 + 2, 0)

        carry = _compute(k0 + 1, 1, carry)
        return carry

    total = lax.fori_loop(0, N_CHUNKS // 2, body, jnp.float32(0.0))
    tot_v[...] = jnp.full((L,), total, dtype=jnp.float32)
    pltpu.sync_copy(tot_v, totals_hbm.at[wid])


@functools.partial(
    pl.kernel,
    out_type=jax.ShapeDtypeStruct((NPAD,), jnp.float32),
    mesh=_mesh,
    compiler_params=_params,
    scratch_types=(
        pltpu.VMEM((NW,), jnp.float32),
        pltpu.VMEM((NW,), jnp.float32),
        pltpu.VMEM((1608,), jnp.int32),
        pltpu.VMEM((NODE_CHUNK,), jnp.int32),
        pltpu.VMEM((NODE_CHUNK,), jnp.int32),
        pltpu.VMEM((NODE_CHUNK,), jnp.float32),
        pltpu.VMEM((NODE_CHUNK,), jnp.float32),
        pltpu.VMEM((NODE_CHUNK,), jnp.float32),
        pltpu.SemaphoreType.DMA,
    ),
)
def _phase2(cums_hbm, tot32_hbm, start_hbm, out_hbm,
            tot_v, bases_v, sv, eidx, bidx, le_v, lb_v, ov, sem):
    wid = _wid()
    pltpu.sync_copy(tot32_hbm, tot_v)
    t0 = tot_v[pl.ds(0, L)]
    t1 = tot_v[pl.ds(L, L)]
    bases_v[pl.ds(0, L)] = plsc.cumsum(t0) - t0
    bases_v[pl.ds(L, L)] = plsc.cumsum(t1) - t1 + jnp.sum(t0)

    def chunk_body(c, _):
        n0 = pl.multiple_of(wid * NODES_PER_W + c * NODE_CHUNK, 8)
        pltpu.sync_copy(start_hbm.at[pl.ds(n0, 1608)], sv)

        def build(j, _):
            lo = sv[pl.ds(j * L, L)]
            hi = sv[pl.ds(j * L + 1, L)]
            eidx[pl.ds(j * L, L)] = jnp.maximum(hi - 1, 0)
            bidx[pl.ds(j * L, L)] = jnp.maximum(lo - 1, 0)
            return 0

        lax.fori_loop(0, NODE_CHUNK // L, build, 0)
        pltpu.async_copy(cums_hbm.at[eidx], le_v, sem).wait()
        pltpu.async_copy(cums_hbm.at[bidx], lb_v, sem).wait()

        def comp(j, _):
            lo = sv[pl.ds(j * L, L)]
            hi = sv[pl.ds(j * L + 1, L)]
            e = hi - 1
            b = lo - 1
            ec = jnp.maximum(e, 0)
            bc = jnp.maximum(b, 0)
            be = plsc.load_gather(bases_v, [ec // SLOTS_PER_W])
            bb = plsc.load_gather(bases_v, [bc // SLOTS_PER_W])
            ge = jnp.where(e < 0, jnp.float32(0.0), le_v[pl.ds(j * L, L)] + be)
            gb = jnp.where(b < 0, jnp.float32(0.0), lb_v[pl.ds(j * L, L)] + bb)
            ov[pl.ds(j * L, L)] = ge - gb
            return 0

        lax.fori_loop(0, NODE_CHUNK // L, comp, 0)
        pltpu.sync_copy(ov, out_hbm.at[pl.ds(n0, NODE_CHUNK)])
        return 0

    lax.fori_loop(0, NODES_PER_W // NODE_CHUNK, chunk_body, 0)


def kernel(net_weights, flat_nodepin, nodepin_start, pin2net_map, num_nodes):
    n = nodepin_start.shape[0] - 1
    p = flat_nodepin.shape[0]
    fnp = flat_nodepin.astype(jnp.int32)
    p2n = pin2net_map.astype(jnp.int32)
    start_pad = jnp.concatenate([
        nodepin_start.astype(jnp.int32),
        jnp.full((SPAD_LEN - (n + 1),), p, dtype=jnp.int32),
    ])
    cums, totals = _phase1(net_weights, fnp, p2n)
    out_pad = _phase2(cums, totals[:, 0], start_pad)
    return out_pad[:n]


# speedup vs baseline: 3290.9040x; 1.0254x over previous
import functools

import jax
import jax.numpy as jnp
from jax import lax
from jax.experimental import pallas as pl
from jax.experimental.pallas import tpu as pltpu
from jax.experimental.pallas import tpu_sc as plsc

NC = 2
NS = 16
NW = NC * NS
L = 16

P = 6_400_000
SLOTS_PER_W = P // NW
CHUNK = 4000
N_CHUNKS = SLOTS_PER_W // CHUNK

NNETS = 100_000

NODES_PER_W = 3200
NPAD = NW * NODES_PER_W
NODE_CHUNK = 1600
SPAD_LEN = (NW - 1) * NODES_PER_W + NODE_CHUNK + 1608

_mesh = plsc.VectorSubcoreMesh(core_axis_name="c", subcore_axis_name="s")
_params = pltpu.CompilerParams(needs_layout_passes=False)


def _wid():
    return lax.axis_index("s") * NC + lax.axis_index("c")


@functools.partial(
    pl.kernel,
    out_type=(
        jax.ShapeDtypeStruct((P,), jnp.float32),
        jax.ShapeDtypeStruct((NW, L), jnp.float32),
    ),
    mesh=_mesh,
    compiler_params=_params,
    scratch_types=(
        pltpu.VMEM((NNETS,), jnp.float32),
        pltpu.VMEM((CHUNK,), jnp.int32),
        pltpu.VMEM((CHUNK,), jnp.int32),
        pltpu.VMEM((CHUNK,), jnp.int32),
        pltpu.VMEM((CHUNK,), jnp.int32),
        pltpu.VMEM((CHUNK,), jnp.float32),
        pltpu.VMEM((CHUNK,), jnp.float32),
        pltpu.VMEM((L,), jnp.float32),
        pltpu.SemaphoreType.DMA,
        pltpu.SemaphoreType.DMA,
        pltpu.SemaphoreType.DMA,
        pltpu.SemaphoreType.DMA,
    ),
)
def _phase1(wt_hbm, fnp_hbm, p2n_hbm, cums_hbm, totals_hbm,
            wt_v, fnp0, fnp1, net0, net1, cums0, cums1, tot_v,
            sem0, sem1, csem0, csem1):
    wid = _wid()
    base = wid * SLOTS_PER_W
    pltpu.sync_copy(wt_hbm, wt_v)

    fnp_b = (fnp0, fnp1)
    net_b = (net0, net1)
    sem_b = (sem0, sem1)
    cums_b = (cums0, cums1)
    csem_b = (csem0, csem1)

    def _chunk_dst(k):
        off = pl.multiple_of(base + k * CHUNK, 8)
        return cums_hbm.at[pl.ds(off, CHUNK)]

    def _prefetch(k, b):
        off = pl.multiple_of(base + k * CHUNK, 8)
        pltpu.sync_copy(fnp_hbm.at[pl.ds(off, CHUNK)], fnp_b[b])
        pltpu.async_copy(p2n_hbm.at[fnp_b[b]], net_b[b], sem_b[b])

    def _compute(k, b, carry):
        pltpu.make_async_copy(p2n_hbm.at[fnp_b[b]], net_b[b], sem_b[b]).wait()
        pltpu.make_async_copy(cums_b[b], _chunk_dst(k), csem_b[b]).wait()

        def vec_body(j, cin):
            idx = net_b[b][pl.ds(j * L, L)]
            w = plsc.load_gather(wt_v, [idx])
            cs = plsc.cumsum(w) + cin
            cums_b[b][pl.ds(j * L, L)] = cs
            return cs[L - 1]

        carry = lax.fori_loop(0, CHUNK // L, vec_body, carry, unroll=4)
        pltpu.async_copy(cums_b[b], _chunk_dst(k), csem_b[b])
        return carry

    pltpu.async_copy(cums0, _chunk_dst(0), csem0)
    pltpu.async_copy(cums1, _chunk_dst(1), csem1)
    _prefetch(0, 0)

    def body(c2, carry):
        k0 = 2 * c2
        _prefetch(k0 + 1, 1)
        carry = _compute(k0, 0, carry)

        @pl.when(c2 < N_CHUNKS // 2 - 1)
        def _():
            _prefetch(k0 + 2, 0)

        carry = _compute(k0 + 1, 1, carry)
        return carry

    total = lax.fori_loop(0, N_CHUNKS // 2, body, jnp.float32(0.0))
    pltpu.make_async_copy(cums0, _chunk_dst(N_CHUNKS - 2), csem0).wait()
    pltpu.make_async_copy(cums1, _chunk_dst(N_CHUNKS - 1), csem1).wait()
    tot_v[...] = jnp.full((L,), total, dtype=jnp.float32)
    pltpu.sync_copy(tot_v, totals_hbm.at[wid])


@functools.partial(
    pl.kernel,
    out_type=jax.ShapeDtypeStruct((NPAD,), jnp.float32),
    mesh=_mesh,
    compiler_params=_params,
    scratch_types=(
        pltpu.VMEM((NW,), jnp.float32),
        pltpu.VMEM((NW,), jnp.float32),
        pltpu.VMEM((1608,), jnp.int32),
        pltpu.VMEM((NODE_CHUNK,), jnp.int32),
        pltpu.VMEM((NODE_CHUNK,), jnp.int32),
        pltpu.VMEM((NODE_CHUNK,), jnp.float32),
        pltpu.VMEM((NODE_CHUNK,), jnp.float32),
        pltpu.VMEM((NODE_CHUNK,), jnp.float32),
        pltpu.SemaphoreType.DMA,
    ),
)
def _phase2(cums_hbm, tot32_hbm, start_hbm, out_hbm,
            tot_v, bases_v, sv, eidx, bidx, le_v, lb_v, ov, sem):
    wid = _wid()
    pltpu.sync_copy(tot32_hbm, tot_v)
    t0 = tot_v[pl.ds(0, L)]
    t1 = tot_v[pl.ds(L, L)]
    bases_v[pl.ds(0, L)] = plsc.cumsum(t0) - t0
    bases_v[pl.ds(L, L)] = plsc.cumsum(t1) - t1 + jnp.sum(t0)

    def chunk_body(c, _):
        n0 = pl.multiple_of(wid * NODES_PER_W + c * NODE_CHUNK, 8)
        pltpu.sync_copy(start_hbm.at[pl.ds(n0, 1608)], sv)

        def build(j, _):
            lo = sv[pl.ds(j * L, L)]
            hi = sv[pl.ds(j * L + 1, L)]
            eidx[pl.ds(j * L, L)] = jnp.maximum(hi - 1, 0)
            bidx[pl.ds(j * L, L)] = jnp.maximum(lo - 1, 0)
            return 0

        lax.fori_loop(0, NODE_CHUNK // L, build, 0)
        pltpu.async_copy(cums_hbm.at[eidx], le_v, sem).wait()
        pltpu.async_copy(cums_hbm.at[bidx], lb_v, sem).wait()

        def comp(j, _):
            lo = sv[pl.ds(j * L, L)]
            hi = sv[pl.ds(j * L + 1, L)]
            e = hi - 1
            b = lo - 1
            ec = jnp.maximum(e, 0)
            bc = jnp.maximum(b, 0)
            be = plsc.load_gather(bases_v, [ec // SLOTS_PER_W])
            bb = plsc.load_gather(bases_v, [bc // SLOTS_PER_W])
            ge = jnp.where(e < 0, jnp.float32(0.0), le_v[pl.ds(j * L, L)] + be)
            gb = jnp.where(b < 0, jnp.float32(0.0), lb_v[pl.ds(j * L, L)] + bb)
            ov[pl.ds(j * L, L)] = ge - gb
            return 0

        lax.fori_loop(0, NODE_CHUNK // L, comp, 0)
        pltpu.sync_copy(ov, out_hbm.at[pl.ds(n0, NODE_CHUNK)])
        return 0

    lax.fori_loop(0, NODES_PER_W // NODE_CHUNK, chunk_body, 0)


def kernel(net_weights, flat_nodepin, nodepin_start, pin2net_map, num_nodes):
    n = nodepin_start.shape[0] - 1
    p = flat_nodepin.shape[0]
    fnp = flat_nodepin.astype(jnp.int32)
    p2n = pin2net_map.astype(jnp.int32)
    start_pad = jnp.concatenate([
        nodepin_start.astype(jnp.int32),
        jnp.full((SPAD_LEN - (n + 1),), p, dtype=jnp.int32),
    ])
    cums, totals = _phase1(net_weights, fnp, p2n)
    out_pad = _phase2(cums, totals[:, 0], start_pad)
    return out_pad[:n]
